# Initial kernel scaffold; baseline (speedup 1.0000x reference)
import jax
import jax.numpy as jnp
from jax.experimental import pallas as pl


def kernel(center, context, negative_samples, center_table, context_table):
    raise NotImplementedError("write your pallas kernel here")


# speedup vs baseline: 1.1849x; 1.1849x over previous
import functools

import jax
import jax.numpy as jnp
from jax import lax
from jax.experimental import pallas as pl
from jax.experimental.pallas import tpu as pltpu
from jax.experimental.pallas import tpu_sc as plsc

B = 16384
D = 64
K = 3
NC = 2
NS = 16
NW = NC * NS
BPW = B // NW
CH = 128
NCH = BPW // CH
L = 16
PIECES = D // L

_mesh = plsc.VectorSubcoreMesh(core_axis_name="c", subcore_axis_name="s")


@functools.partial(
    pl.kernel,
    mesh=_mesh,
    compiler_params=pltpu.CompilerParams(use_tc_tiling_on_sc=False),
    out_type=[
        jax.ShapeDtypeStruct((B, L), jnp.float32),
        jax.ShapeDtypeStruct((K * B, L), jnp.float32),
    ],
    scratch_types=[
        pltpu.VMEM((NCH, CH), jnp.int32),
        pltpu.VMEM((NCH, CH), jnp.int32),
        pltpu.VMEM((K * NCH, CH), jnp.int32),
        pltpu.VMEM((CH, D), jnp.float32),
        pltpu.VMEM((CH, D), jnp.float32),
        pltpu.VMEM((K, CH, D), jnp.float32),
        pltpu.VMEM((BPW, L), jnp.float32),
        pltpu.VMEM((K, BPW, L), jnp.float32),
        pltpu.SemaphoreType.DMA,
    ],
)
def _sc_scores(center_hbm, context_hbm, negt_hbm, ctab_hbm, xtab_hbm,
               pos_out, neg_out,
               cidx, xidx, nidx, crows, xrows, nrows, pbuf, nbuf, sem):
    wid = lax.axis_index("s") * NC + lax.axis_index("c")
    base = wid * BPW

    for j in range(NCH):
        pltpu.sync_copy(center_hbm.at[pl.ds(base + j * CH, CH)], cidx.at[j])
        pltpu.sync_copy(context_hbm.at[pl.ds(base + j * CH, CH)], xidx.at[j])
        for k in range(K):
            pltpu.sync_copy(negt_hbm.at[pl.ds(k * B + base + j * CH, CH)],
                            nidx.at[k * NCH + j])

    for j in range(NCH):
        cps = [
            pltpu.async_copy(ctab_hbm.at[cidx.at[j]], crows, sem),
            pltpu.async_copy(xtab_hbm.at[xidx.at[j]], xrows, sem),
        ]
        for k in range(K):
            cps.append(pltpu.async_copy(xtab_hbm.at[nidx.at[k * NCH + j]],
                                        nrows.at[k], sem))
        for cp in cps:
            cp.wait()

        def body(e, carry, j=j):
            cs = [crows[e, pl.ds(p * L, L)] for p in range(PIECES)]
            xs = [xrows[e, pl.ds(p * L, L)] for p in range(PIECES)]
            pv = cs[0] * xs[0] + cs[1] * xs[1] + cs[2] * xs[2] + cs[3] * xs[3]
            pbuf[j * CH + e] = pv
            for k in range(K):
                ns = [nrows[k, e, pl.ds(p * L, L)] for p in range(PIECES)]
                nv = (cs[0] * ns[0] + cs[1] * ns[1]
                      + cs[2] * ns[2] + cs[3] * ns[3])
                nbuf[k, j * CH + e] = nv
            return carry

        lax.fori_loop(0, CH, body, 0)

    pltpu.sync_copy(pbuf, pos_out.at[pl.ds(base, BPW)])
    for k in range(K):
        pltpu.sync_copy(nbuf.at[k], neg_out.at[pl.ds(k * B + base, BPW)])


def _loss_body(pos_ref, neg_ref, out_ref):
    pos = jnp.sum(pos_ref[...], axis=1)
    neg = jnp.sum(neg_ref[...], axis=1)

    def softplus(z):
        return jnp.maximum(z, 0.0) + jnp.log1p(jnp.exp(-jnp.abs(z)))

    total = (jnp.sum(softplus(-pos)) + jnp.sum(softplus(neg))) / B
    out_ref[...] = jnp.reshape(total, (1, 1))


_loss = pl.pallas_call(
    _loss_body,
    out_shape=jax.ShapeDtypeStruct((1, 1), jnp.float32),
)


def kernel(center, context, negative_samples, center_table, context_table):
    center = center.astype(jnp.int32)
    context = context.astype(jnp.int32)
    negt = negative_samples.astype(jnp.int32).T.reshape(-1)
    pos_pv, neg_pv = _sc_scores(center, context, negt,
                                center_table, context_table)
    loss = _loss(pos_pv, neg_pv)
    return loss[0, 0]
